# baseline (speedup 1.0000x reference)
import functools

import jax
import jax.numpy as jnp
from jax import lax
from jax.experimental import pallas as pl
from jax.experimental.pallas import tpu as pltpu
from jax.experimental.pallas import tpu_sc as plsc

N_NODES = 100000
NSEG = 64
HID = 128
OUT = 128

NC = 2
NS = 16
NW = NC * NS

BLK = 128
NBLK = N_NODES // BLK
TAIL = N_NODES - NBLK * BLK
TAIL_BASE = NBLK * BLK
NB_MIN = NBLK // NW
NB_EXTRA = NBLK - NB_MIN * NW
EXTRA_BASE = NB_MIN * NW
NCHUNK = NB_MIN // 2
CHROWS = 2 * BLK
NBANK = 4


def _sc_segment_sums(x, idxmat):
    mesh = plsc.VectorSubcoreMesh(core_axis_name="c", subcore_axis_name="s")

    @functools.partial(
        pl.kernel,
        out_type=(
            jax.ShapeDtypeStruct((NC, NBANK, NSEG, HID), jnp.float32),
            jax.ShapeDtypeStruct((NW, NSEG, 16), jnp.float32),
        ),
        mesh=mesh,
        scratch_types=[
            pltpu.VMEM((NB_MIN + 2, BLK), jnp.int32),
            pltpu.VMEM((TAIL,), jnp.int32),
            pltpu.VMEM((CHROWS, HID), jnp.float32),
            pltpu.VMEM((CHROWS, HID), jnp.float32),
            pltpu.VMEM((CHROWS, HID), jnp.float32),
            pltpu.VMEM((8, HID), jnp.float32),
            pltpu.VMEM((NSEG + 1, 16), jnp.float32),
            pltpu.VMEM_SHARED((NBANK, NSEG, HID), jnp.float32),
            pltpu.SemaphoreType.DMA,
            pltpu.SemaphoreType.DMA,
            pltpu.SemaphoreType.DMA,
            pltpu.SemaphoreType.DMA,
            pltpu.SemaphoreType.DMA,
            pltpu.SemaphoreType.DMA,
        ],
    )
    def k(x_hbm, idx_hbm, out_hbm, outc_hbm, idxall, tidxbuf, xbuf0, xbuf1,
          xbuf2, zbuf, cnt, acc, sem0, sem1, sem2, ssem0, ssem1, ssem2):
        cid = lax.axis_index("c")
        sid = lax.axis_index("s")
        wid = sid * NC + cid

        lo = wid * NB_MIN

        xbufs = (xbuf0, xbuf1, xbuf2)
        sems = (sem0, sem1, sem2)
        ssems = (ssem0, ssem1, ssem2)
        NBUF = 3

        idx_desc = pltpu.async_copy(
            idx_hbm.at[pl.ds(lo, NB_MIN)], idxall.at[pl.ds(0, NB_MIN)],
            sem0)

        zero = jnp.zeros((16,), jnp.float32)

        def zero_cnt(r, _):
            cnt[r] = zero
            return 0

        lax.fori_loop(0, NSEG + 1, zero_cnt, 0)

        bank = lax.rem(sid, NBANK)

        @pl.when(sid == 0)
        def _():
            for r in range(8):
                for c in range(HID // 16):
                    zbuf[r, pl.ds(c * 16, 16)] = zero
            for bk in range(NBANK):
                for r in range(NSEG // 8):
                    pltpu.sync_copy(zbuf, acc.at[bk, pl.ds(r * 8, 8)])

        idx_desc.wait()

        def issue(c):
            return pltpu.async_copy(
                x_hbm.at[pl.ds((lo + 2 * c) * BLK, CHROWS)],
                xbufs[c % NBUF], sems[c % NBUF])

        dma = [issue(0), issue(1), None]
        sct = [None, None, None]
        plsc.subcore_barrier()

        for c in range(NCHUNK):
            s = c % NBUF
            dma[s].wait()
            buf = xbufs[s]
            d0 = pltpu.make_async_copy(buf.at[pl.ds(0, BLK)],
                                       acc.at[bank].at[idxall.at[2 * c]], ssems[s])
            d1 = pltpu.make_async_copy(buf.at[pl.ds(BLK, BLK)],
                                       acc.at[bank].at[idxall.at[2 * c + 1]],
                                       ssems[s])
            d0.start(add=True)
            d1.start(add=True)
            sct[s] = (d0, d1)
            if c + 2 < NCHUNK:
                t = (c + 2) % NBUF
                if sct[t] is not None:
                    sct[t][0].wait()
                    sct[t][1].wait()
                    sct[t] = None
                dma[t] = issue(c + 2)

        for s in range(NBUF):
            if sct[s] is not None:
                sct[s][0].wait()
                sct[s][1].wait()

        @pl.when(wid < NB_EXTRA)
        def _():
            b = EXTRA_BASE + wid
            pltpu.sync_copy(idx_hbm.at[b], idxall.at[NB_MIN])
            pltpu.sync_copy(x_hbm.at[pl.ds(b * BLK, BLK)],
                            xbuf0.at[pl.ds(0, BLK)])
            pltpu.sync_copy(xbuf0.at[pl.ds(0, BLK)],
                            acc.at[bank].at[idxall.at[NB_MIN]], add=True)

        @pl.when(wid == NW - 1)
        def _():
            pltpu.sync_copy(idx_hbm.at[NBLK], idxall.at[NB_MIN + 1])
            pltpu.sync_copy(idx_hbm.at[NBLK, pl.ds(0, TAIL)], tidxbuf)
            pltpu.sync_copy(x_hbm.at[pl.ds(TAIL_BASE, TAIL)],
                            xbuf1.at[pl.ds(0, TAIL)])
            pltpu.sync_copy(xbuf1.at[pl.ds(0, TAIL)], acc.at[bank].at[tidxbuf],
                            add=True)

        def count_block(b, _):
            valid = (b < NB_MIN) | ((b == NB_MIN) & (wid < NB_EXTRA)) | (
                (b == NB_MIN + 1) & (wid == NW - 1))

            @pl.when(valid)
            def _():
                first = idxall[b, pl.ds(0, 16)][0]
                last = idxall[b, pl.ds(BLK - 16, 16)][15]

                @pl.when(first == last)
                def _():
                    cnt[first] = cnt[first] + jnp.full((16,), float(BLK),
                                                       jnp.float32)

                @pl.when(first != last)
                def _():
                    for kk in range(BLK // 16):
                        idv = idxall[b, pl.ds(kk * 16, 16)]
                        cf = idv[0]
                        cl = idv[15]

                        @pl.when(cf == cl)
                        def _(cf=cf):
                            cnt[cf] = cnt[cf] + jnp.full((16,), 16.0,
                                                         jnp.float32)

                        @pl.when(cf != cl)
                        def _(idv=idv):
                            one = jnp.full((16,), 1.0, jnp.float32)
                            for r in range(16):
                                seg = idv[r]
                                cnt[seg] = cnt[seg] + one
            return 0

        lax.fori_loop(0, NB_MIN + 2, count_block, 0)

        pltpu.sync_copy(cnt.at[pl.ds(0, NSEG)], outc_hbm.at[wid])

        plsc.subcore_barrier()

        @pl.when(sid == 0)
        def _():
            pltpu.sync_copy(acc, out_hbm.at[cid])

    return k(x, idxmat)


def _tc_finish(partials, counts, W1, b1, gamma, beta, W2, b2):
    def body(p_ref, c_ref, w1_ref, b1_ref, g_ref, be_ref, w2_ref, b2_ref,
             o_ref):
        sums = jnp.sum(p_ref[...], axis=(0, 1))
        cnt = jnp.sum(c_ref[...], axis=0)[:, 0]
        pooled = sums / jnp.maximum(cnt, 1.0)[:, None]
        h = jnp.dot(pooled, w1_ref[...],
                    preferred_element_type=jnp.float32,
                    precision=lax.Precision.HIGHEST) + b1_ref[...]
        mean = jnp.mean(h, axis=0, keepdims=True)
        var = jnp.mean((h - mean) * (h - mean), axis=0, keepdims=True)
        h = (h - mean) * lax.rsqrt(var + 1e-5) * g_ref[...] + be_ref[...]
        h = jnp.maximum(h, 0.0)
        o_ref[...] = jnp.dot(h, w2_ref[...],
                             preferred_element_type=jnp.float32,
                             precision=lax.Precision.HIGHEST) + b2_ref[...]

    return pl.pallas_call(
        body,
        out_shape=jax.ShapeDtypeStruct((NSEG, OUT), jnp.float32),
    )(partials, counts, W1, b1.reshape(1, HID), gamma.reshape(1, HID),
      beta.reshape(1, HID), W2, b2.reshape(1, OUT))


def kernel(x, edge_index, edge_attr, u, batch, W1, b1, gamma, beta, W2, b2):
    del edge_index, edge_attr, u
    batch_i32 = batch.astype(jnp.int32)
    pad = (NBLK + 1) * BLK - N_NODES
    idxmat = jnp.concatenate(
        [batch_i32, jnp.full((pad,), NSEG, jnp.int32)]
    ).reshape(NBLK + 1, BLK)

    partials, counts = _sc_segment_sums(x, idxmat)
    return _tc_finish(partials, counts, W1, b1, gamma, beta, W2, b2)


# speedup vs baseline: 1.2297x; 1.2297x over previous
import functools

import jax
import jax.numpy as jnp
from jax import lax
from jax.experimental import pallas as pl
from jax.experimental.pallas import tpu as pltpu
from jax.experimental.pallas import tpu_sc as plsc

N_NODES = 100000
NSEG = 64
HID = 128
OUT = 128

NC = 2
NS = 16
NW = NC * NS

BLK = 128
NBLK = N_NODES // BLK
TAIL = N_NODES - NBLK * BLK
TAIL_BASE = NBLK * BLK
NB_MIN = NBLK // NW
NB_EXTRA = NBLK - NB_MIN * NW
EXTRA_BASE = NB_MIN * NW
NCHUNK = NB_MIN // 2
CHROWS = 2 * BLK
NBANK = 1


def _sc_segment_sums(x, idxmat):
    mesh = plsc.VectorSubcoreMesh(core_axis_name="c", subcore_axis_name="s")

    @functools.partial(
        pl.kernel,
        out_type=(
            jax.ShapeDtypeStruct((NC, NBANK, NSEG, HID), jnp.float32),
            jax.ShapeDtypeStruct((NW, NSEG, HID), jnp.float32),
            jax.ShapeDtypeStruct((NW, NSEG, 16), jnp.float32),
        ),
        mesh=mesh,
        scratch_types=[
            pltpu.VMEM((NB_MIN + 2, BLK), jnp.int32),
            pltpu.VMEM((TAIL,), jnp.int32),
            pltpu.VMEM((CHROWS, HID), jnp.float32),
            pltpu.VMEM((CHROWS, HID), jnp.float32),
            pltpu.VMEM((8, HID), jnp.float32),
            pltpu.VMEM((NSEG + 1, 16), jnp.float32),
            pltpu.VMEM((NSEG, HID), jnp.float32),
            pltpu.VMEM_SHARED((NBANK, NSEG, HID), jnp.float32),
            pltpu.SemaphoreType.DMA,
            pltpu.SemaphoreType.DMA,
        ],
    )
    def k(x_hbm, idx_hbm, out_hbm, outl_hbm, outc_hbm, idxall, tidxbuf,
          xbuf0, xbuf1, zbuf, cnt, accl, acc, sem0, sem1):
        cid = lax.axis_index("c")
        sid = lax.axis_index("s")
        wid = sid * NC + cid

        lo = wid * NB_MIN

        xbufs = (xbuf0, xbuf1)
        sems = (sem0, sem1)
        NBUF = 2

        idx_desc = pltpu.async_copy(
            idx_hbm.at[pl.ds(lo, NB_MIN)], idxall.at[pl.ds(0, NB_MIN)],
            sem0)

        zero = jnp.zeros((16,), jnp.float32)

        def zero_cnt(r, _):
            cnt[r] = zero
            return 0

        lax.fori_loop(0, NSEG + 1, zero_cnt, 0)

        def zero_accl(r, _):
            for col in range(HID // 16):
                accl[r, pl.ds(col * 16, 16)] = zero
            return 0

        lax.fori_loop(0, NSEG, zero_accl, 0)

        bank = lax.rem(sid, NBANK)

        @pl.when(sid == 0)
        def _():
            for r in range(8):
                for c in range(HID // 16):
                    zbuf[r, pl.ds(c * 16, 16)] = zero
            for bk in range(NBANK):
                for r in range(NSEG // 8):
                    pltpu.sync_copy(zbuf, acc.at[bk, pl.ds(r * 8, 8)])

        idx_desc.wait()

        def issue(c):
            return pltpu.async_copy(
                x_hbm.at[pl.ds((lo + 2 * c) * BLK, CHROWS)],
                xbufs[c % NBUF], sems[c % NBUF])

        def accum_rows(xref, base, seg):
            def grp(g, carry):
                accs = list(carry)
                for r in range(8):
                    row = base + g * 8 + r
                    for col in range(8):
                        accs[col] = accs[col] + xref[row,
                                                     pl.ds(col * 16, 16)]
                return tuple(accs)

            init = tuple(jnp.zeros((16,), jnp.float32) for _ in range(8))
            accs = lax.fori_loop(0, BLK // 8, grp, init)
            for col in range(8):
                accl[seg, pl.ds(col * 16, 16)] = (
                    accl[seg, pl.ds(col * 16, 16)] + accs[col])

        dma = [issue(0), None]
        plsc.subcore_barrier()

        for c in range(NCHUNK):
            s = c % NBUF
            dma[s].wait()
            if c + 1 < NCHUNK:
                dma[(c + 1) % NBUF] = issue(c + 1)
            buf = xbufs[s]
            for half in range(2):
                b = 2 * c + half
                first = idxall[b, pl.ds(0, 16)][0]
                last = idxall[b, pl.ds(BLK - 16, 16)][15]

                @pl.when(first == last)
                def _(buf=buf, half=half, first=first):
                    accum_rows(buf, half * BLK, first)

                @pl.when(first != last)
                def _(buf=buf, half=half, b=b):
                    pltpu.sync_copy(buf.at[pl.ds(half * BLK, BLK)],
                                    acc.at[bank].at[idxall.at[b]],
                                    add=True)

        @pl.when(wid < NB_EXTRA)
        def _():
            b = EXTRA_BASE + wid
            pltpu.sync_copy(idx_hbm.at[b], idxall.at[NB_MIN])
            pltpu.sync_copy(x_hbm.at[pl.ds(b * BLK, BLK)],
                            xbuf0.at[pl.ds(0, BLK)])
            pltpu.sync_copy(xbuf0.at[pl.ds(0, BLK)],
                            acc.at[bank].at[idxall.at[NB_MIN]], add=True)

        @pl.when(wid == NW - 1)
        def _():
            pltpu.sync_copy(idx_hbm.at[NBLK], idxall.at[NB_MIN + 1])
            pltpu.sync_copy(idx_hbm.at[NBLK, pl.ds(0, TAIL)], tidxbuf)
            pltpu.sync_copy(x_hbm.at[pl.ds(TAIL_BASE, TAIL)],
                            xbuf1.at[pl.ds(0, TAIL)])
            pltpu.sync_copy(xbuf1.at[pl.ds(0, TAIL)], acc.at[bank].at[tidxbuf],
                            add=True)

        def count_block(b, _):
            valid = (b < NB_MIN) | ((b == NB_MIN) & (wid < NB_EXTRA)) | (
                (b == NB_MIN + 1) & (wid == NW - 1))

            @pl.when(valid)
            def _():
                first = idxall[b, pl.ds(0, 16)][0]
                last = idxall[b, pl.ds(BLK - 16, 16)][15]

                @pl.when(first == last)
                def _():
                    cnt[first] = cnt[first] + jnp.full((16,), float(BLK),
                                                       jnp.float32)

                @pl.when(first != last)
                def _():
                    for kk in range(BLK // 16):
                        idv = idxall[b, pl.ds(kk * 16, 16)]
                        cf = idv[0]
                        cl = idv[15]

                        @pl.when(cf == cl)
                        def _(cf=cf):
                            cnt[cf] = cnt[cf] + jnp.full((16,), 16.0,
                                                         jnp.float32)

                        @pl.when(cf != cl)
                        def _(idv=idv):
                            one = jnp.full((16,), 1.0, jnp.float32)
                            for r in range(16):
                                seg = idv[r]
                                cnt[seg] = cnt[seg] + one
            return 0

        lax.fori_loop(0, NB_MIN + 2, count_block, 0)

        pltpu.sync_copy(cnt.at[pl.ds(0, NSEG)], outc_hbm.at[wid])
        pltpu.sync_copy(accl, outl_hbm.at[wid])

        plsc.subcore_barrier()

        @pl.when(sid == 0)
        def _():
            pltpu.sync_copy(acc, out_hbm.at[cid])

    return k(x, idxmat)


def _tc_finish(partials, locals_, counts, W1, b1, gamma, beta, W2, b2):
    def body(p_ref, l_ref, c_ref, w1_ref, b1_ref, g_ref, be_ref, w2_ref,
             b2_ref, o_ref):
        sums = (jnp.sum(p_ref[...], axis=(0, 1))
                + jnp.sum(l_ref[...], axis=0))
        cnt = jnp.sum(c_ref[...], axis=0)[:, 0]
        pooled = sums / jnp.maximum(cnt, 1.0)[:, None]
        h = jnp.dot(pooled, w1_ref[...],
                    preferred_element_type=jnp.float32,
                    precision=lax.Precision.HIGHEST) + b1_ref[...]
        mean = jnp.mean(h, axis=0, keepdims=True)
        var = jnp.mean((h - mean) * (h - mean), axis=0, keepdims=True)
        h = (h - mean) * lax.rsqrt(var + 1e-5) * g_ref[...] + be_ref[...]
        h = jnp.maximum(h, 0.0)
        o_ref[...] = jnp.dot(h, w2_ref[...],
                             preferred_element_type=jnp.float32,
                             precision=lax.Precision.HIGHEST) + b2_ref[...]

    return pl.pallas_call(
        body,
        out_shape=jax.ShapeDtypeStruct((NSEG, OUT), jnp.float32),
    )(partials, locals_, counts, W1, b1.reshape(1, HID),
      gamma.reshape(1, HID), beta.reshape(1, HID), W2, b2.reshape(1, OUT))


def kernel(x, edge_index, edge_attr, u, batch, W1, b1, gamma, beta, W2, b2):
    del edge_index, edge_attr, u
    batch_i32 = batch.astype(jnp.int32)
    pad = (NBLK + 1) * BLK - N_NODES
    idxmat = jnp.concatenate(
        [batch_i32, jnp.full((pad,), NSEG, jnp.int32)]
    ).reshape(NBLK + 1, BLK)

    partials, locals_, counts = _sc_segment_sums(x, idxmat)
    return _tc_finish(partials, locals_, counts, W1, b1, gamma, beta, W2,
                      b2)


# speedup vs baseline: 1.3579x; 1.1043x over previous
import functools

import jax
import jax.numpy as jnp
from jax import lax
from jax.experimental import pallas as pl
from jax.experimental.pallas import tpu as pltpu
from jax.experimental.pallas import tpu_sc as plsc

N_NODES = 100000
NSEG = 64
HID = 128
OUT = 128

NC = 2
NS = 16
NW = NC * NS

BLK = 128
NBLK = N_NODES // BLK
TAIL = N_NODES - NBLK * BLK
TAIL_BASE = NBLK * BLK
NB_MIN = NBLK // NW
NB_EXTRA = NBLK - NB_MIN * NW
EXTRA_BASE = NB_MIN * NW
NCHUNK = NB_MIN // 2
CHROWS = 2 * BLK
NBANK = 1


def _sc_segment_sums(x, idxmat):
    mesh = plsc.VectorSubcoreMesh(core_axis_name="c", subcore_axis_name="s")

    @functools.partial(
        pl.kernel,
        out_type=(
            jax.ShapeDtypeStruct((NC, NBANK, NSEG, HID), jnp.float32),
            jax.ShapeDtypeStruct((NW, NSEG, HID), jnp.float32),
            jax.ShapeDtypeStruct((NW, NSEG, 16), jnp.float32),
        ),
        mesh=mesh,
        scratch_types=[
            pltpu.VMEM((NB_MIN + 2, BLK), jnp.int32),
            pltpu.VMEM((TAIL,), jnp.int32),
            pltpu.VMEM((CHROWS, HID), jnp.float32),
            pltpu.VMEM((CHROWS, HID), jnp.float32),
            pltpu.VMEM((CHROWS, HID), jnp.float32),
            pltpu.VMEM((8, HID), jnp.float32),
            pltpu.VMEM((NSEG + 1, 16), jnp.float32),
            pltpu.VMEM((NSEG, HID), jnp.float32),
            pltpu.VMEM_SHARED((NBANK, NSEG, HID), jnp.float32),
            pltpu.SemaphoreType.DMA,
            pltpu.SemaphoreType.DMA,
            pltpu.SemaphoreType.DMA,
        ],
    )
    def k(x_hbm, idx_hbm, out_hbm, outl_hbm, outc_hbm, idxall, tidxbuf,
          xbuf0, xbuf1, xbuf2, zbuf, cnt, accl, acc, sem0, sem1, sem2):
        cid = lax.axis_index("c")
        sid = lax.axis_index("s")
        wid = sid * NC + cid

        lo = wid * NB_MIN

        xbufs = (xbuf0, xbuf1, xbuf2)
        sems = (sem0, sem1, sem2)
        NBUF = 3

        idx_desc = pltpu.async_copy(
            idx_hbm.at[pl.ds(lo, NB_MIN)], idxall.at[pl.ds(0, NB_MIN)],
            sem0)

        zero = jnp.zeros((16,), jnp.float32)

        def zero_cnt(r, _):
            cnt[r] = zero
            return 0

        lax.fori_loop(0, NSEG + 1, zero_cnt, 0)

        def zero_accl(r, _):
            for col in range(HID // 16):
                accl[r, pl.ds(col * 16, 16)] = zero
            return 0

        lax.fori_loop(0, NSEG, zero_accl, 0)

        bank = lax.rem(sid, NBANK)

        @pl.when(sid == 0)
        def _():
            for r in range(8):
                for c in range(HID // 16):
                    zbuf[r, pl.ds(c * 16, 16)] = zero
            for bk in range(NBANK):
                for r in range(NSEG // 8):
                    pltpu.sync_copy(zbuf, acc.at[bk, pl.ds(r * 8, 8)])

        idx_desc.wait()

        def issue(c):
            return pltpu.async_copy(
                x_hbm.at[pl.ds((lo + 2 * c) * BLK, CHROWS)],
                xbufs[c % NBUF], sems[c % NBUF])

        def accum_rows(xref, base, seg):
            def grp(g, carry):
                accs = list(carry)
                for r in range(8):
                    row = base + g * 8 + r
                    for col in range(8):
                        accs[col] = accs[col] + xref[row,
                                                     pl.ds(col * 16, 16)]
                return tuple(accs)

            init = tuple(jnp.zeros((16,), jnp.float32) for _ in range(8))
            accs = lax.fori_loop(0, BLK // 8, grp, init)
            for col in range(8):
                accl[seg, pl.ds(col * 16, 16)] = (
                    accl[seg, pl.ds(col * 16, 16)] + accs[col])

        dma = [issue(0), issue(1), None]
        plsc.subcore_barrier()

        for c in range(NCHUNK):
            s = c % NBUF
            dma[s].wait()
            if c + 2 < NCHUNK:
                dma[(c + 2) % NBUF] = issue(c + 2)
            buf = xbufs[s]
            for half in range(2):
                b = 2 * c + half
                first = idxall[b, pl.ds(0, 16)][0]
                last = idxall[b, pl.ds(BLK - 16, 16)][15]

                @pl.when(first == last)
                def _(buf=buf, half=half, first=first):
                    accum_rows(buf, half * BLK, first)

                @pl.when(first != last)
                def _(buf=buf, half=half, b=b):
                    pltpu.sync_copy(buf.at[pl.ds(half * BLK, BLK)],
                                    acc.at[bank].at[idxall.at[b]],
                                    add=True)

        @pl.when(wid < NB_EXTRA)
        def _():
            b = EXTRA_BASE + wid
            pltpu.sync_copy(idx_hbm.at[b], idxall.at[NB_MIN])
            pltpu.sync_copy(x_hbm.at[pl.ds(b * BLK, BLK)],
                            xbuf0.at[pl.ds(0, BLK)])
            pltpu.sync_copy(xbuf0.at[pl.ds(0, BLK)],
                            acc.at[bank].at[idxall.at[NB_MIN]], add=True)

        @pl.when(wid == NW - 1)
        def _():
            pltpu.sync_copy(idx_hbm.at[NBLK], idxall.at[NB_MIN + 1])
            pltpu.sync_copy(idx_hbm.at[NBLK, pl.ds(0, TAIL)], tidxbuf)
            pltpu.sync_copy(x_hbm.at[pl.ds(TAIL_BASE, TAIL)],
                            xbuf1.at[pl.ds(0, TAIL)])
            pltpu.sync_copy(xbuf1.at[pl.ds(0, TAIL)], acc.at[bank].at[tidxbuf],
                            add=True)

        def count_block(b, _):
            valid = (b < NB_MIN) | ((b == NB_MIN) & (wid < NB_EXTRA)) | (
                (b == NB_MIN + 1) & (wid == NW - 1))

            @pl.when(valid)
            def _():
                first = idxall[b, pl.ds(0, 16)][0]
                last = idxall[b, pl.ds(BLK - 16, 16)][15]

                @pl.when(first == last)
                def _():
                    cnt[first] = cnt[first] + jnp.full((16,), float(BLK),
                                                       jnp.float32)

                @pl.when(first != last)
                def _():
                    for kk in range(BLK // 16):
                        idv = idxall[b, pl.ds(kk * 16, 16)]
                        cf = idv[0]
                        cl = idv[15]

                        @pl.when(cf == cl)
                        def _(cf=cf):
                            cnt[cf] = cnt[cf] + jnp.full((16,), 16.0,
                                                         jnp.float32)

                        @pl.when(cf != cl)
                        def _(idv=idv):
                            one = jnp.full((16,), 1.0, jnp.float32)
                            for r in range(16):
                                seg = idv[r]
                                cnt[seg] = cnt[seg] + one
            return 0

        lax.fori_loop(0, NB_MIN + 2, count_block, 0)

        pltpu.sync_copy(cnt.at[pl.ds(0, NSEG)], outc_hbm.at[wid])
        pltpu.sync_copy(accl, outl_hbm.at[wid])

        plsc.subcore_barrier()

        @pl.when(sid == 0)
        def _():
            pltpu.sync_copy(acc, out_hbm.at[cid])

    return k(x, idxmat)


def _tc_finish(partials, locals_, counts, W1, b1, gamma, beta, W2, b2):
    def body(p_ref, l_ref, c_ref, w1_ref, b1_ref, g_ref, be_ref, w2_ref,
             b2_ref, o_ref):
        sums = (jnp.sum(p_ref[...], axis=(0, 1))
                + jnp.sum(l_ref[...], axis=0))
        cnt = jnp.sum(c_ref[...], axis=0)[:, 0]
        pooled = sums / jnp.maximum(cnt, 1.0)[:, None]
        h = jnp.dot(pooled, w1_ref[...],
                    preferred_element_type=jnp.float32,
                    precision=lax.Precision.HIGHEST) + b1_ref[...]
        mean = jnp.mean(h, axis=0, keepdims=True)
        var = jnp.mean((h - mean) * (h - mean), axis=0, keepdims=True)
        h = (h - mean) * lax.rsqrt(var + 1e-5) * g_ref[...] + be_ref[...]
        h = jnp.maximum(h, 0.0)
        o_ref[...] = jnp.dot(h, w2_ref[...],
                             preferred_element_type=jnp.float32,
                             precision=lax.Precision.HIGHEST) + b2_ref[...]

    return pl.pallas_call(
        body,
        out_shape=jax.ShapeDtypeStruct((NSEG, OUT), jnp.float32),
    )(partials, locals_, counts, W1, b1.reshape(1, HID),
      gamma.reshape(1, HID), beta.reshape(1, HID), W2, b2.reshape(1, OUT))


def kernel(x, edge_index, edge_attr, u, batch, W1, b1, gamma, beta, W2, b2):
    del edge_index, edge_attr, u
    batch_i32 = batch.astype(jnp.int32)
    pad = (NBLK + 1) * BLK - N_NODES
    idxmat = jnp.concatenate(
        [batch_i32, jnp.full((pad,), NSEG, jnp.int32)]
    ).reshape(NBLK + 1, BLK)

    partials, locals_, counts = _sc_segment_sums(x, idxmat)
    return _tc_finish(partials, locals_, counts, W1, b1, gamma, beta, W2,
                      b2)


# speedup vs baseline: 1.3846x; 1.0196x over previous
import functools

import jax
import jax.numpy as jnp
from jax import lax
from jax.experimental import pallas as pl
from jax.experimental.pallas import tpu as pltpu
from jax.experimental.pallas import tpu_sc as plsc

N_NODES = 100000
NSEG = 64
HID = 128
OUT = 128

NC = 2
NS = 16
NW = NC * NS

BLK = 128
NBLK = N_NODES // BLK
TAIL = N_NODES - NBLK * BLK
TAIL_BASE = NBLK * BLK
NB_MIN = NBLK // NW
NB_EXTRA = NBLK - NB_MIN * NW
EXTRA_BASE = NB_MIN * NW
NCHUNK = NB_MIN // 2
CHROWS = 2 * BLK
NBANK = 1


def _sc_segment_sums(x, idxmat):
    mesh = plsc.VectorSubcoreMesh(core_axis_name="c", subcore_axis_name="s")

    @functools.partial(
        pl.kernel,
        out_type=(
            jax.ShapeDtypeStruct((NC, NBANK, NSEG, HID), jnp.float32),
            jax.ShapeDtypeStruct((NW, NSEG, HID), jnp.float32),
            jax.ShapeDtypeStruct((NW, NSEG, 16), jnp.float32),
        ),
        mesh=mesh,
        scratch_types=[
            pltpu.VMEM((NB_MIN + 2, BLK), jnp.int32),
            pltpu.VMEM((TAIL,), jnp.int32),
            pltpu.VMEM((CHROWS, HID), jnp.float32),
            pltpu.VMEM((CHROWS, HID), jnp.float32),
            pltpu.VMEM((CHROWS, HID), jnp.float32),
            pltpu.VMEM((4, HID), jnp.float32),
            pltpu.VMEM((NSEG + 1, 16), jnp.float32),
            pltpu.VMEM((NSEG, HID), jnp.float32),
            pltpu.VMEM_SHARED((NBANK, NSEG, HID), jnp.float32),
            pltpu.SemaphoreType.DMA,
            pltpu.SemaphoreType.DMA,
            pltpu.SemaphoreType.DMA,
        ],
    )
    def k(x_hbm, idx_hbm, out_hbm, outl_hbm, outc_hbm, idxall, tidxbuf,
          xbuf0, xbuf1, xbuf2, zbuf, cnt, accl, acc, sem0, sem1, sem2):
        cid = lax.axis_index("c")
        sid = lax.axis_index("s")
        wid = sid * NC + cid

        lo = wid * NB_MIN

        xbufs = (xbuf0, xbuf1, xbuf2)
        sems = (sem0, sem1, sem2)
        NBUF = 3

        idx_desc = pltpu.async_copy(
            idx_hbm.at[pl.ds(lo, NB_MIN)], idxall.at[pl.ds(0, NB_MIN)],
            sem2)

        zero = jnp.zeros((16,), jnp.float32)

        def zero_cnt(r, _):
            cnt[r] = zero
            return 0

        lax.fori_loop(0, NSEG + 1, zero_cnt, 0)

        def zero_accl(r, _):
            for col in range(HID // 16):
                accl[r, pl.ds(col * 16, 16)] = zero
            return 0

        lax.fori_loop(0, NSEG, zero_accl, 0)

        bank = lax.rem(sid, NBANK)

        def issue(c):
            return pltpu.async_copy(
                x_hbm.at[pl.ds((lo + 2 * c) * BLK, CHROWS)],
                xbufs[c % NBUF], sems[c % NBUF])

        prime = [issue(0), issue(1)]

        for r in range(4):
            for c in range(HID // 16):
                zbuf[r, pl.ds(c * 16, 16)] = zero
        pltpu.sync_copy(zbuf, acc.at[0, pl.ds(sid * 4, 4)])

        idx_desc.wait()

        def accum_rows(xref, base, seg):
            def grp(g, carry):
                accs = list(carry)
                for r in range(8):
                    row = base + g * 8 + r
                    for col in range(8):
                        accs[col] = accs[col] + xref[row,
                                                     pl.ds(col * 16, 16)]
                return tuple(accs)

            init = tuple(jnp.zeros((16,), jnp.float32) for _ in range(8))
            accs = lax.fori_loop(0, BLK // 8, grp, init)
            for col in range(8):
                accl[seg, pl.ds(col * 16, 16)] = (
                    accl[seg, pl.ds(col * 16, 16)] + accs[col])

        dma = [prime[0], prime[1], None]
        plsc.subcore_barrier()

        for c in range(NCHUNK):
            s = c % NBUF
            dma[s].wait()
            if c + 2 < NCHUNK:
                dma[(c + 2) % NBUF] = issue(c + 2)
            buf = xbufs[s]
            for half in range(2):
                b = 2 * c + half
                first = idxall[b, pl.ds(0, 16)][0]
                last = idxall[b, pl.ds(BLK - 16, 16)][15]

                @pl.when(first == last)
                def _(buf=buf, half=half, first=first):
                    accum_rows(buf, half * BLK, first)

                @pl.when(first != last)
                def _(buf=buf, half=half, b=b):
                    pltpu.sync_copy(buf.at[pl.ds(half * BLK, BLK)],
                                    acc.at[bank].at[idxall.at[b]],
                                    add=True)

        @pl.when(wid < NB_EXTRA)
        def _():
            b = EXTRA_BASE + wid
            pltpu.sync_copy(idx_hbm.at[b], idxall.at[NB_MIN])
            pltpu.sync_copy(x_hbm.at[pl.ds(b * BLK, BLK)],
                            xbuf0.at[pl.ds(0, BLK)])
            pltpu.sync_copy(xbuf0.at[pl.ds(0, BLK)],
                            acc.at[bank].at[idxall.at[NB_MIN]], add=True)

        @pl.when(wid == NW - 1)
        def _():
            pltpu.sync_copy(idx_hbm.at[NBLK], idxall.at[NB_MIN + 1])
            pltpu.sync_copy(idx_hbm.at[NBLK, pl.ds(0, TAIL)], tidxbuf)
            pltpu.sync_copy(x_hbm.at[pl.ds(TAIL_BASE, TAIL)],
                            xbuf1.at[pl.ds(0, TAIL)])
            pltpu.sync_copy(xbuf1.at[pl.ds(0, TAIL)], acc.at[bank].at[tidxbuf],
                            add=True)

        def count_block(b, _):
            valid = (b < NB_MIN) | ((b == NB_MIN) & (wid < NB_EXTRA)) | (
                (b == NB_MIN + 1) & (wid == NW - 1))

            @pl.when(valid)
            def _():
                first = idxall[b, pl.ds(0, 16)][0]
                last = idxall[b, pl.ds(BLK - 16, 16)][15]

                @pl.when(first == last)
                def _():
                    cnt[first] = cnt[first] + jnp.full((16,), float(BLK),
                                                       jnp.float32)

                @pl.when(first != last)
                def _():
                    for kk in range(BLK // 16):
                        idv = idxall[b, pl.ds(kk * 16, 16)]
                        cf = idv[0]
                        cl = idv[15]

                        @pl.when(cf == cl)
                        def _(cf=cf):
                            cnt[cf] = cnt[cf] + jnp.full((16,), 16.0,
                                                         jnp.float32)

                        @pl.when(cf != cl)
                        def _(idv=idv):
                            one = jnp.full((16,), 1.0, jnp.float32)
                            for r in range(16):
                                seg = idv[r]
                                cnt[seg] = cnt[seg] + one
            return 0

        lax.fori_loop(0, NB_MIN + 2, count_block, 0)

        pltpu.sync_copy(cnt.at[pl.ds(0, NSEG)], outc_hbm.at[wid])
        pltpu.sync_copy(accl, outl_hbm.at[wid])

        plsc.subcore_barrier()

        @pl.when(sid == 0)
        def _():
            pltpu.sync_copy(acc, out_hbm.at[cid])

    return k(x, idxmat)


def _tc_finish(partials, locals_, counts, W1, b1, gamma, beta, W2, b2):
    def body(p_ref, l_ref, c_ref, w1_ref, b1_ref, g_ref, be_ref, w2_ref,
             b2_ref, o_ref):
        sums = (jnp.sum(p_ref[...], axis=(0, 1))
                + jnp.sum(l_ref[...], axis=0))
        cnt = jnp.sum(c_ref[...], axis=0)[:, 0]
        pooled = sums / jnp.maximum(cnt, 1.0)[:, None]
        h = jnp.dot(pooled, w1_ref[...],
                    preferred_element_type=jnp.float32,
                    precision=lax.Precision.HIGHEST) + b1_ref[...]
        mean = jnp.mean(h, axis=0, keepdims=True)
        var = jnp.mean((h - mean) * (h - mean), axis=0, keepdims=True)
        h = (h - mean) * lax.rsqrt(var + 1e-5) * g_ref[...] + be_ref[...]
        h = jnp.maximum(h, 0.0)
        o_ref[...] = jnp.dot(h, w2_ref[...],
                             preferred_element_type=jnp.float32,
                             precision=lax.Precision.HIGHEST) + b2_ref[...]

    return pl.pallas_call(
        body,
        out_shape=jax.ShapeDtypeStruct((NSEG, OUT), jnp.float32),
    )(partials, locals_, counts, W1, b1.reshape(1, HID),
      gamma.reshape(1, HID), beta.reshape(1, HID), W2, b2.reshape(1, OUT))


def kernel(x, edge_index, edge_attr, u, batch, W1, b1, gamma, beta, W2, b2):
    del edge_index, edge_attr, u
    batch_i32 = batch.astype(jnp.int32)
    pad = (NBLK + 1) * BLK - N_NODES
    idxmat = jnp.concatenate(
        [batch_i32, jnp.full((pad,), NSEG, jnp.int32)]
    ).reshape(NBLK + 1, BLK)

    partials, locals_, counts = _sc_segment_sums(x, idxmat)
    return _tc_finish(partials, locals_, counts, W1, b1, gamma, beta, W2,
                      b2)
